# Initial kernel scaffold; baseline (speedup 1.0000x reference)
import functools

import jax
import jax.numpy as jnp
from jax import lax
from jax.experimental import pallas as pl
from jax.experimental.pallas import tpu as pltpu
from jax.experimental.pallas import tpu_sc as plsc

_SMOOTHING = 0.1
_CONFIDENCE = 1.0 - _SMOOTHING
_PAD = 0

_NC = 2
_NS = 16
_NW = _NC * _NS
_LANES = 16
_CHUNK = 128


def _sc_gather_build(n_rows, n_cols):
    rpw = n_rows // _NW
    nch = rpw // _CHUNK
    mesh = plsc.VectorSubcoreMesh(core_axis_name="c", subcore_axis_name="s")

    @functools.partial(
        pl.kernel,
        out_type=jax.ShapeDtypeStruct((_NW, 128), jnp.float32),
        mesh=mesh,
        scratch_types=[
            pltpu.VMEM((rpw,), jnp.int32),
            pltpu.VMEM((nch, _CHUNK), jnp.int32),
            pltpu.VMEM((nch, _CHUNK), jnp.float32),
            pltpu.VMEM((128,), jnp.float32),
            pltpu.SemaphoreType.DMA,
        ],
    )
    def sc_gather(xflat_hbm, tgt_hbm, out_hbm, tgt_v, idx_v, vals_v, row_v, sem):
        wid = lax.axis_index("s") * _NC + lax.axis_index("c")
        base = wid * rpw
        pltpu.sync_copy(tgt_hbm.at[pl.ds(base, rpw)], tgt_v)
        per_row = _CHUNK // _LANES
        for c in range(rpw // _LANES):
            t = tgt_v[pl.ds(c * _LANES, _LANES)]
            row = base + c * _LANES + lax.iota(jnp.int32, _LANES)
            idx_v[c // per_row, pl.ds((c % per_row) * _LANES, _LANES)] = (
                row * n_cols + t)
        for h in range(nch):
            pltpu.async_copy(xflat_hbm.at[idx_v.at[h]], vals_v.at[h], sem).wait()
        acc = jnp.zeros((_LANES,), jnp.float32)
        for c in range(rpw // _LANES):
            t = tgt_v[pl.ds(c * _LANES, _LANES)]
            v = vals_v[c // per_row, pl.ds((c % per_row) * _LANES, _LANES)]
            acc = acc + jnp.where(t != _PAD, v, 0.0)
        for j in range(128 // _LANES):
            row_v[pl.ds(j * _LANES, _LANES)] = jnp.zeros((_LANES,), jnp.float32)
        row_v[pl.ds(0, _LANES)] = acc
        pltpu.sync_copy(row_v, out_hbm.at[wid])

    return sc_gather


def _tc_body(n_blocks, fill, c1, x_ref, t_ref, g_ref, out_ref):
    i = pl.program_id(0)
    xb = x_ref[...]
    t = t_ref[0, 0, :]
    row_sum = jnp.sum(xb, axis=1)
    col0 = xb[:, 0]
    contrib = jnp.where(t != _PAD, c1 - fill * (row_sum - col0), 0.0)
    partial = jnp.sum(contrib)

    @pl.when(i == 0)
    def _init():
        out_ref[0, 0] = 0.0

    out_ref[0, 0] += partial

    @pl.when(i == n_blocks - 1)
    def _finish():
        out_ref[0, 0] += -(_CONFIDENCE - fill) * jnp.sum(g_ref[...])


def kernel(x, target):
    n, v = x.shape
    fill = _SMOOTHING / (v - 2)
    c1 = float(_CONFIDENCE * jnp.log(_CONFIDENCE) + _SMOOTHING * jnp.log(fill))
    t32 = target.astype(jnp.int32)

    g_partials = _sc_gather_build(n, v)(x.reshape(-1), t32)

    r = 64
    n_blocks = n // r
    total = pl.pallas_call(
        functools.partial(_tc_body, n_blocks, fill, c1),
        grid=(n_blocks,),
        in_specs=[
            pl.BlockSpec((r, v), lambda i: (i, 0)),
            pl.BlockSpec((1, 1, r), lambda i: (i, 0, 0)),
            pl.BlockSpec((_NW, 128), lambda i: (0, 0)),
        ],
        out_specs=pl.BlockSpec(memory_space=pltpu.SMEM),
        out_shape=jax.ShapeDtypeStruct((1, 1), jnp.float32),
        compiler_params=pltpu.CompilerParams(
            dimension_semantics=("arbitrary",)),
    )(x, t32.reshape(n_blocks, 1, r), g_partials)
    return total[0, 0]


# speedup vs baseline: 3.4478x; 3.4478x over previous
import functools
import math

import jax
import jax.numpy as jnp
from jax import lax
from jax.experimental import pallas as pl
from jax.experimental.pallas import tpu as pltpu
from jax.experimental.pallas import tpu_sc as plsc

_SMOOTHING = 0.1
_CONFIDENCE = 1.0 - _SMOOTHING
_PAD = 0

_NC = 2
_NS = 16
_NW = _NC * _NS
_LANES = 16
_CHUNK = 128


def _sc_gather_build(n_rows, n_cols):
    rpw = n_rows // _NW
    nch = rpw // _CHUNK
    mesh = plsc.VectorSubcoreMesh(core_axis_name="c", subcore_axis_name="s")

    @functools.partial(
        pl.kernel,
        out_type=jax.ShapeDtypeStruct((_NW, 128), jnp.float32),
        mesh=mesh,
        scratch_types=[
            pltpu.VMEM((rpw,), jnp.int32),
            pltpu.VMEM((nch, _CHUNK), jnp.int32),
            pltpu.VMEM((nch, _CHUNK), jnp.float32),
            pltpu.VMEM((128,), jnp.float32),
            pltpu.SemaphoreType.DMA,
        ],
    )
    def sc_gather(xflat_hbm, tgt_hbm, out_hbm, tgt_v, idx_v, vals_v, row_v, sem):
        wid = lax.axis_index("s") * _NC + lax.axis_index("c")
        base = wid * rpw
        pltpu.sync_copy(tgt_hbm.at[pl.ds(base, rpw)], tgt_v)
        per_row = _CHUNK // _LANES
        for c in range(rpw // _LANES):
            t = tgt_v[pl.ds(c * _LANES, _LANES)]
            row = base + c * _LANES + lax.iota(jnp.int32, _LANES)
            idx_v[c // per_row, pl.ds((c % per_row) * _LANES, _LANES)] = (
                row * n_cols + t)
        for h in range(nch):
            pltpu.async_copy(xflat_hbm.at[idx_v.at[h]], vals_v.at[h], sem).wait()
        acc = jnp.zeros((_LANES,), jnp.float32)
        for c in range(rpw // _LANES):
            t = tgt_v[pl.ds(c * _LANES, _LANES)]
            v = vals_v[c // per_row, pl.ds((c % per_row) * _LANES, _LANES)]
            acc = acc + jnp.where(t != _PAD, v, 0.0)
        for j in range(128 // _LANES):
            row_v[pl.ds(j * _LANES, _LANES)] = jnp.zeros((_LANES,), jnp.float32)
        row_v[pl.ds(0, _LANES)] = acc
        pltpu.sync_copy(row_v, out_hbm.at[wid])

    return sc_gather


def _tc_body(n_blocks, fill, c1, x_ref, t_ref, g_ref, out_ref):
    i = pl.program_id(0)
    xb = x_ref[...]
    t = t_ref[0, 0, :]
    row_sum = jnp.sum(xb, axis=1)
    col0 = xb[:, 0]
    contrib = jnp.where(t != _PAD, c1 - fill * (row_sum - col0), 0.0)
    partial = jnp.sum(contrib)

    @pl.when(i == 0)
    def _init():
        out_ref[0, 0] = 0.0

    out_ref[0, 0] += partial

    @pl.when(i == n_blocks - 1)
    def _finish():
        out_ref[0, 0] += -(_CONFIDENCE - fill) * jnp.sum(g_ref[...])


def kernel(x, target):
    n, v = x.shape
    fill = _SMOOTHING / (v - 2)
    c1 = _CONFIDENCE * math.log(_CONFIDENCE) + _SMOOTHING * math.log(fill)
    t32 = target.astype(jnp.int32)

    g_partials = _sc_gather_build(n, v)(x.reshape(-1), t32)

    r = 64
    n_blocks = n // r
    total = pl.pallas_call(
        functools.partial(_tc_body, n_blocks, fill, c1),
        grid=(n_blocks,),
        in_specs=[
            pl.BlockSpec((r, v), lambda i: (i, 0)),
            pl.BlockSpec((1, 1, r), lambda i: (i, 0, 0)),
            pl.BlockSpec((_NW, 128), lambda i: (0, 0)),
        ],
        out_specs=pl.BlockSpec(memory_space=pltpu.SMEM),
        out_shape=jax.ShapeDtypeStruct((1, 1), jnp.float32),
        compiler_params=pltpu.CompilerParams(
            dimension_semantics=("arbitrary",)),
    )(x, t32.reshape(n_blocks, 1, r), g_partials)
    return total[0, 0]


# speedup vs baseline: 3.6326x; 1.0536x over previous
import functools
import math

import jax
import jax.numpy as jnp
from jax import lax
from jax.experimental import pallas as pl
from jax.experimental.pallas import tpu as pltpu
from jax.experimental.pallas import tpu_sc as plsc

_SMOOTHING = 0.1
_CONFIDENCE = 1.0 - _SMOOTHING
_PAD = 0

_NC = 2
_NS = 16
_NW = _NC * _NS
_LANES = 16
_CHUNK = 128


def _sc_gather_build(n_rows, n_cols):
    rpw = n_rows // _NW
    nch = rpw // _CHUNK
    mesh = plsc.VectorSubcoreMesh(core_axis_name="c", subcore_axis_name="s")

    @functools.partial(
        pl.kernel,
        out_type=jax.ShapeDtypeStruct((_NW, 128), jnp.float32),
        mesh=mesh,
        scratch_types=[
            pltpu.VMEM((rpw,), jnp.int32),
            pltpu.VMEM((nch, _CHUNK), jnp.int32),
            pltpu.VMEM((nch, _CHUNK), jnp.float32),
            pltpu.VMEM((128,), jnp.float32),
            pltpu.SemaphoreType.DMA,
        ],
    )
    def sc_gather(xflat_hbm, tgt_hbm, out_hbm, tgt_v, idx_v, vals_v, row_v, sem):
        wid = lax.axis_index("s") * _NC + lax.axis_index("c")
        base = wid * rpw
        pltpu.sync_copy(tgt_hbm.at[pl.ds(base, rpw)], tgt_v)
        per_row = _CHUNK // _LANES
        for c in range(rpw // _LANES):
            t = tgt_v[pl.ds(c * _LANES, _LANES)]
            row = base + c * _LANES + lax.iota(jnp.int32, _LANES)
            idx_v[c // per_row, pl.ds((c % per_row) * _LANES, _LANES)] = (
                row * n_cols + t)
        for h in range(nch):
            pltpu.async_copy(xflat_hbm.at[idx_v.at[h]], vals_v.at[h], sem).wait()
        acc = jnp.zeros((_LANES,), jnp.float32)
        for c in range(rpw // _LANES):
            t = tgt_v[pl.ds(c * _LANES, _LANES)]
            v = vals_v[c // per_row, pl.ds((c % per_row) * _LANES, _LANES)]
            acc = acc + jnp.where(t != _PAD, v, 0.0)
        for j in range(128 // _LANES):
            row_v[pl.ds(j * _LANES, _LANES)] = jnp.zeros((_LANES,), jnp.float32)
        row_v[pl.ds(0, _LANES)] = acc
        pltpu.sync_copy(row_v, out_hbm.at[wid])

    return sc_gather


def _tc_body(n_blocks, fill, c1, x_ref, t_ref, g_ref, out_ref):
    i = pl.program_id(0)
    xb = x_ref[...]
    t = t_ref[0, 0, :]
    row_sum = jnp.sum(xb, axis=1)
    col0 = xb[:, 0]
    contrib = jnp.where(t != _PAD, c1 - fill * (row_sum - col0), 0.0)
    partial = jnp.sum(contrib)

    @pl.when(i == 0)
    def _init():
        out_ref[0, 0] = 0.0

    out_ref[0, 0] += partial

    @pl.when(i == n_blocks - 1)
    def _finish():
        out_ref[0, 0] += -(_CONFIDENCE - fill) * jnp.sum(g_ref[...])


def kernel(x, target):
    n, v = x.shape
    fill = _SMOOTHING / (v - 2)
    c1 = _CONFIDENCE * math.log(_CONFIDENCE) + _SMOOTHING * math.log(fill)
    t32 = target.astype(jnp.int32)

    g_partials = _sc_gather_build(n, v)(x.reshape(-1), t32)

    r = 256
    n_blocks = n // r
    total = pl.pallas_call(
        functools.partial(_tc_body, n_blocks, fill, c1),
        grid=(n_blocks,),
        in_specs=[
            pl.BlockSpec((r, v), lambda i: (i, 0)),
            pl.BlockSpec((1, 1, r), lambda i: (i, 0, 0)),
            pl.BlockSpec((_NW, 128), lambda i: (0, 0)),
        ],
        out_specs=pl.BlockSpec(memory_space=pltpu.SMEM),
        out_shape=jax.ShapeDtypeStruct((1, 1), jnp.float32),
        compiler_params=pltpu.CompilerParams(
            dimension_semantics=("arbitrary",)),
    )(x, t32.reshape(n_blocks, 1, r), g_partials)
    return total[0, 0]
